# baseline (speedup 1.0000x reference)
import functools

import jax
import jax.numpy as jnp
from jax import lax
from jax.experimental import pallas as pl
from jax.experimental.pallas import tpu as pltpu
from jax.experimental.pallas import tpu_sc as plsc

_K = 1024


def _vq_idx_kernel(x_ref, w_ref, c_ref, idx_ref, loss_ref):
    nb, pp, cc = x_ref.shape
    xp = x_ref[...].reshape(nb * pp, cc)
    w = w_ref[...]
    s = jax.lax.dot_general(
        xp, w, (((1,), (1,)), ((), ())),
        preferred_element_type=jnp.float32)
    a = jnp.sum(xp * xp, axis=1, keepdims=True)
    d = (a - 2.0 * s) + c_ref[...]
    m = jnp.min(d, axis=1, keepdims=True)
    cols = jax.lax.broadcasted_iota(jnp.int32, d.shape, 1)
    idxc = jnp.min(jnp.where(d == m, cols, _K), axis=1, keepdims=True)
    idx_ref[...] = idxc.reshape(nb, pp, 1)

    @pl.when(pl.program_id(0) == 0)
    def _():
        loss_ref[...] = jnp.zeros_like(loss_ref)

    loss_ref[...] += jnp.sum(m, keepdims=True)


def _tc_half(xt_half, weight, cvec):
    bb, p, c = xt_half.shape
    return pl.pallas_call(
        _vq_idx_kernel,
        grid=(bb // 4,),
        in_specs=[
            pl.BlockSpec((4, p, c), lambda i: (i, 0, 0)),
            pl.BlockSpec((_K, c), lambda i: (0, 0)),
            pl.BlockSpec((1, _K), lambda i: (0, 0)),
        ],
        out_specs=[
            pl.BlockSpec((4, p, 1), lambda i: (i, 0, 0)),
            pl.BlockSpec((1, 1), lambda i: (0, 0)),
        ],
        out_shape=[
            jax.ShapeDtypeStruct((bb, p, 1), jnp.int32),
            jax.ShapeDtypeStruct((1, 1), jnp.float32),
        ],
    )(xt_half, weight, cvec)


def _sc_gather(table, idx):
    n, d_ = idx.shape[0], table.shape[1]
    info = plsc.get_sparse_core_info()
    nw = info.num_cores * info.num_subcores
    b_per_w = n // nw
    chunk = min(b_per_w, 256)
    mesh = plsc.VectorSubcoreMesh(core_axis_name="c", subcore_axis_name="s")

    @functools.partial(
        pl.kernel, mesh=mesh,
        out_type=jax.ShapeDtypeStruct((n, d_), jnp.float32),
        scratch_types=[
            pltpu.VMEM((chunk,), jnp.int32),
            pltpu.VMEM((chunk, d_), jnp.float32),
            pltpu.SemaphoreType.DMA,
        ],
    )
    def k(table_hbm, idx_hbm, out_hbm, idx_v, rows_v, sem):
        wid = lax.axis_index("s") * info.num_cores + lax.axis_index("c")
        for j in range(b_per_w // chunk):
            base = wid * b_per_w + j * chunk
            pltpu.sync_copy(idx_hbm.at[pl.ds(base, chunk)], idx_v)
            pltpu.async_copy(table_hbm.at[idx_v], rows_v, sem).wait()
            pltpu.sync_copy(rows_v, out_hbm.at[pl.ds(base, chunk)])

    return k(table, idx)


def kernel(x, weight, decay, commitment_cost):
    b, c, h, w_ = x.shape
    p = h * w_
    hb = b // 2
    cvec = jnp.sum(weight**2, axis=1).reshape(1, _K)
    halves = []
    losses = []
    idxs = []
    for lo in (0, hb):
        xt = jnp.transpose(
            x[lo:lo + hb], (0, 2, 3, 1)).reshape(hb, p, c)
        idx_h, loss_h = _tc_half(xt, weight, cvec)
        qt_h = _sc_gather(weight, idx_h.reshape(hb * p))
        quant_h = jnp.transpose(qt_h.reshape(hb, h, w_, c), (0, 3, 1, 2))
        halves.append(quant_h)
        losses.append(loss_h)
        idxs.append(idx_h)
    quantized = jnp.concatenate(halves, axis=0)
    embed_idx = jnp.concatenate(idxs, axis=0).reshape(b, h, w_)
    loss_sum = losses[0][0, 0] + losses[1][0, 0]
    latent_loss = commitment_cost * (loss_sum / x.size)
    return (quantized, latent_loss, embed_idx)


# speedup vs baseline: 1.5762x; 1.5762x over previous
import jax
import jax.numpy as jnp
from jax.experimental import pallas as pl

_K = 1024


def _vq_kernel(x_ref, w_ref, c_ref, q_ref, idx_ref, loss_ref):
    nb, pp, cc = x_ref.shape
    xp = x_ref[...].reshape(nb * pp, cc)
    w = w_ref[...]
    s = jax.lax.dot_general(
        xp, w, (((1,), (1,)), ((), ())),
        preferred_element_type=jnp.float32)
    a = jnp.sum(xp * xp, axis=1, keepdims=True)
    d = (a - 2.0 * s) + c_ref[...]
    m = jnp.min(d, axis=1, keepdims=True)
    cols = jax.lax.broadcasted_iota(jnp.int32, d.shape, 1)
    idxc = jnp.min(jnp.where(d == m, cols, _K), axis=1, keepdims=True)
    idx_ref[...] = idxc.reshape(nb, pp, 1)
    oh = (cols == idxc).astype(jnp.float32)
    q = jnp.dot(oh, w, preferred_element_type=jnp.float32)
    q_ref[...] = q.reshape(nb, pp, cc)

    @pl.when(pl.program_id(0) == 0)
    def _():
        loss_ref[...] = jnp.zeros_like(loss_ref)

    loss_ref[...] += jnp.sum(m, keepdims=True)


def kernel(x, weight, decay, commitment_cost):
    b, c, h, w_ = x.shape
    p = h * w_
    xt = jnp.transpose(x, (0, 2, 3, 1)).reshape(b, p, c)
    cvec = jnp.sum(weight**2, axis=1).reshape(1, _K)
    q, idx, loss = pl.pallas_call(
        _vq_kernel,
        grid=(b // 4,),
        in_specs=[
            pl.BlockSpec((4, p, c), lambda i: (i, 0, 0)),
            pl.BlockSpec((_K, c), lambda i: (0, 0)),
            pl.BlockSpec((1, _K), lambda i: (0, 0)),
        ],
        out_specs=[
            pl.BlockSpec((4, p, c), lambda i: (i, 0, 0)),
            pl.BlockSpec((4, p, 1), lambda i: (i, 0, 0)),
            pl.BlockSpec((1, 1), lambda i: (0, 0)),
        ],
        out_shape=[
            jax.ShapeDtypeStruct((b, p, c), jnp.float32),
            jax.ShapeDtypeStruct((b, p, 1), jnp.int32),
            jax.ShapeDtypeStruct((1, 1), jnp.float32),
        ],
    )(xt, weight, cvec)
    quantized = jnp.transpose(q.reshape(b, h, w_, c), (0, 3, 1, 2))
    embed_idx = idx.reshape(b, h, w_)
    latent_loss = commitment_cost * (loss[0, 0] / x.size)
    return (quantized, latent_loss, embed_idx)
